# baseline (speedup 1.0000x reference)
import jax
import jax.numpy as jnp
from jax.experimental import pallas as pl
from jax.experimental.pallas import tpu as pltpu

_NT = 512
_KT = 8192
_L = 128

_TDIMS = (((1,), (1,)), ((), ()))


def _vq_loss_kernel(maxlen_ref, x_ref, c_ref, out_ref, acc_ref, csq_ref,
                    cbf_ref):
    i = pl.program_id(0)
    j = pl.program_id(1)
    nk = pl.num_programs(1)
    max_ilen = maxlen_ref[0]
    t_dim = maxlen_ref[1]

    t0 = (i * _NT) % t_dim
    tile_active = t0 < max_ilen

    @pl.when(jnp.logical_and(i == 0, j == 0))
    def _init_out():
        out_ref[0, 0] = 0.0

    @pl.when(tile_active)
    def _compute():
        @pl.when(i == 0)
        def _fill_c():
            cb = c_ref[...].astype(jnp.bfloat16)
            cbf_ref[pl.ds(j * _KT, _KT), :] = cb
            ones = jnp.ones((1, cb.shape[1]), dtype=jnp.bfloat16)
            csq_ref[:, pl.ds(j * _KT, _KT)] = jax.lax.dot_general(
                ones, cb * cb, _TDIMS, preferred_element_type=jnp.float32)

        x = (-2.0 * x_ref[...]).astype(jnp.bfloat16)
        c = cbf_ref[pl.ds(j * _KT, _KT), :]

        part = jax.lax.dot_general(
            x, c, _TDIMS, preferred_element_type=jnp.float32)
        csq = csq_ref[:, pl.ds(j * _KT, _KT)]

        m = part[:, 0:_L] + csq[:, 0:_L]
        for g in range(1, _KT // _L):
            sl = slice(g * _L, (g + 1) * _L)
            m = jnp.minimum(m, part[:, sl] + csq[:, sl])

        @pl.when(j == 0)
        def _first():
            acc_ref[...] = m

        @pl.when(j != 0)
        def _rest():
            acc_ref[...] = jnp.minimum(acc_ref[...], m)

        @pl.when(j == nk - 1)
        def _finish():
            x32 = x_ref[...]
            x_sq = jnp.sum(x32 * x32, axis=1, keepdims=True)
            minv = jnp.min(acc_ref[...], axis=1, keepdims=True) + x_sq
            t_local = t0 + jax.lax.broadcasted_iota(jnp.int32, (_NT, 1), 0)
            masked = jnp.where(t_local < max_ilen, minv, 0.0)
            out_ref[0, 0] += jnp.sum(masked)


def kernel(xs, ilens, codebook):
    b, t, d = xs.shape
    k = codebook.shape[0]
    n = b * t
    flat = xs.reshape(n, d)

    max_ilen = jnp.max(ilens)
    scalars = jnp.stack([max_ilen, jnp.int32(t)])

    total = pl.pallas_call(
        _vq_loss_kernel,
        grid=(n // _NT, k // _KT),
        in_specs=[
            pl.BlockSpec(memory_space=pltpu.SMEM),
            pl.BlockSpec((_NT, d), lambda i, j: (i, 0)),
            pl.BlockSpec((_KT, d), lambda i, j: (j, 0)),
        ],
        out_specs=pl.BlockSpec((1, 1), lambda i, j: (0, 0),
                               memory_space=pltpu.SMEM),
        out_shape=jax.ShapeDtypeStruct((1, 1), jnp.float32),
        scratch_shapes=[
            pltpu.VMEM((_NT, _L), jnp.float32),
            pltpu.VMEM((1, k), jnp.float32),
            pltpu.VMEM((k, d), jnp.bfloat16),
        ],
        compiler_params=pltpu.CompilerParams(
            dimension_semantics=("arbitrary", "arbitrary")),
    )(scalars, flat, codebook)

    count = jnp.float32(b * d) * max_ilen.astype(jnp.float32)
    commit_loss = total[0, 0] / count
    loss = 0.25 * commit_loss
    return (loss, commit_loss)


# speedup vs baseline: 1.2077x; 1.2077x over previous
import jax
import jax.numpy as jnp
from jax.experimental import pallas as pl
from jax.experimental.pallas import tpu as pltpu

_NT = 1024
_KT = 8192
_L = 128

_TDIMS = (((1,), (1,)), ((), ()))


def kernel(xs, ilens, codebook):
    b, t, d = xs.shape
    k = codebook.shape[0]
    n = b * t
    flat = xs.reshape(n, d)
    ni = n // _NT
    nk = k // _KT

    def body(ilens_ref, x_ref, c_ref, out_ref, acc_ref, csq_ref, cbf_ref):
        i = pl.program_id(0)
        j = pl.program_id(1)

        max_ilen = ilens_ref[0]
        for z in range(1, b):
            max_ilen = jnp.maximum(max_ilen, ilens_ref[z])

        t0 = (i * _NT) % t
        tile_active = t0 < max_ilen

        @pl.when(jnp.logical_and(i == 0, j == 0))
        def _init_out():
            out_ref[0, 0] = 0.0

        @pl.when(tile_active)
        def _compute():
            @pl.when(i == 0)
            def _fill_c():
                cb = c_ref[...].astype(jnp.bfloat16)
                cbf_ref[pl.ds(j * _KT, _KT), :] = cb
                ones = jnp.ones((1, cb.shape[1]), dtype=jnp.bfloat16)
                csq_ref[:, pl.ds(j * _KT, _KT)] = jax.lax.dot_general(
                    ones, cb * cb, _TDIMS, preferred_element_type=jnp.float32)

            x = (-2.0 * x_ref[...]).astype(jnp.bfloat16)
            c = cbf_ref[pl.ds(j * _KT, _KT), :]

            part = jax.lax.dot_general(
                x, c, _TDIMS, preferred_element_type=jnp.float32)
            csq = csq_ref[:, pl.ds(j * _KT, _KT)]

            m = part[:, 0:_L] + csq[:, 0:_L]
            for g in range(1, _KT // _L):
                sl = slice(g * _L, (g + 1) * _L)
                m = jnp.minimum(m, part[:, sl] + csq[:, sl])

            @pl.when(j == 0)
            def _first():
                acc_ref[...] = m

            @pl.when(j != 0)
            def _rest():
                acc_ref[...] = jnp.minimum(acc_ref[...], m)

            @pl.when(j == nk - 1)
            def _finish():
                x32 = x_ref[...]
                x_sq = jnp.sum(x32 * x32, axis=1, keepdims=True)
                minv = jnp.min(acc_ref[...], axis=1, keepdims=True) + x_sq
                t_local = t0 + jax.lax.broadcasted_iota(
                    jnp.int32, (_NT, 1), 0)
                masked = jnp.where(t_local < max_ilen, minv, 0.0)
                out_ref[0, 0] += jnp.sum(masked)

        @pl.when(jnp.logical_and(i == ni - 1, j == nk - 1))
        def _finalize():
            count = jnp.float32(b * d) * max_ilen.astype(jnp.float32)
            commit = out_ref[0, 0] / count
            out_ref[0, 0] = 0.25 * commit
            out_ref[0, 1] = commit

    res = pl.pallas_call(
        body,
        grid=(ni, nk),
        in_specs=[
            pl.BlockSpec(memory_space=pltpu.SMEM),
            pl.BlockSpec((_NT, d), lambda i, j: (i, 0)),
            pl.BlockSpec((_KT, d), lambda i, j: (j, 0)),
        ],
        out_specs=pl.BlockSpec((1, 2), lambda i, j: (0, 0),
                               memory_space=pltpu.SMEM),
        out_shape=jax.ShapeDtypeStruct((1, 2), jnp.float32),
        scratch_shapes=[
            pltpu.VMEM((_NT, _L), jnp.float32),
            pltpu.VMEM((1, k), jnp.float32),
            pltpu.VMEM((k, d), jnp.bfloat16),
        ],
        compiler_params=pltpu.CompilerParams(
            dimension_semantics=("arbitrary", "arbitrary")),
    )(ilens, flat, codebook)

    return (res[0, 0], res[0, 1])
